# Initial kernel scaffold; baseline (speedup 1.0000x reference)
import jax
import jax.numpy as jnp
from jax.experimental import pallas as pl


def kernel(x, edge_index, W_in, b_in, W1, b1, g1, beta1, W2, b2, g2, beta2, W3, b3, g3, beta3, W_out, b_out):
    raise NotImplementedError("write your pallas kernel here")


# speedup vs baseline: 4.2668x; 4.2668x over previous
import functools

import jax
import jax.numpy as jnp
from jax import lax
from jax.experimental import pallas as pl
from jax.experimental.pallas import tpu as pltpu
from jax.experimental.pallas import tpu_sc as plsc

N = 10000
E = 160000
D_IN = 256
D_H = 512
D_OUT = 8

NC, NS, L = 2, 16, 16
NPAD = 10240
ROWS_PT = NPAD // NS
CHUNK = 128
EPT = 10240
NCHUNK = EPT // CHUNK
EPAD = NS * EPT
CB = D_H // 128
RB = 400
NRB = N // RB
EPS = 1e-5

_mesh = plsc.VectorSubcoreMesh(core_axis_name="c", subcore_axis_name="s")


@functools.partial(
    pl.kernel,
    out_type=jax.ShapeDtypeStruct((NPAD,), jnp.float32),
    mesh=_mesh,
    scratch_types=[
        pltpu.VMEM_SHARED((NPAD,), jnp.float32),
        pltpu.VMEM((ROWS_PT,), jnp.float32),
        pltpu.VMEM((NCHUNK, CHUNK), jnp.int32),
    ],
)
def _sc_degree(dst_hbm, deg_hbm, hist, ones_v, idx_v):
    c = lax.axis_index("c")
    s = lax.axis_index("s")

    @pl.when(c == 0)
    def _():
        def fill(i, carry):
            ones_v[pl.ds(i * L, L)] = jnp.ones((L,), jnp.float32)
            return carry

        lax.fori_loop(0, ROWS_PT // L, fill, 0)
        pltpu.sync_copy(dst_hbm.at[s], idx_v)
        pltpu.sync_copy(ones_v, hist.at[pl.ds(s * ROWS_PT, ROWS_PT)])
        plsc.subcore_barrier()

        def body(j, carry):
            pltpu.sync_copy(ones_v.at[pl.ds(0, CHUNK)],
                            hist.at[idx_v.at[j]], add=True)
            return carry

        lax.fori_loop(0, NCHUNK, body, 0)
        plsc.subcore_barrier()
        pltpu.sync_copy(hist.at[pl.ds(s * ROWS_PT, ROWS_PT)],
                        deg_hbm.at[pl.ds(s * ROWS_PT, ROWS_PT)])


@functools.partial(
    pl.kernel,
    out_type=jax.ShapeDtypeStruct((CB, NPAD, 128), jnp.float32),
    mesh=_mesh,
    scratch_types=[
        pltpu.VMEM_SHARED((NPAD, 128), jnp.float32),
        pltpu.VMEM((NCHUNK, CHUNK), jnp.int32),
        pltpu.VMEM((NCHUNK, CHUNK), jnp.int32),
        pltpu.VMEM((CHUNK, 128), jnp.float32),
    ],
)
def _sc_aggregate(tab_hbm, src_hbm, dst_hbm, agg_hbm, acc, idx_s, idx_d, gbuf):
    c = lax.axis_index("c")
    s = lax.axis_index("s")
    pltpu.sync_copy(src_hbm.at[s], idx_s)
    pltpu.sync_copy(dst_hbm.at[s], idx_d)
    for cbl in range(CB // NC):
        cb = c * (CB // NC) + cbl
        tab = tab_hbm.at[cb]
        pltpu.sync_copy(tab.at[pl.ds(s * ROWS_PT, ROWS_PT)],
                        acc.at[pl.ds(s * ROWS_PT, ROWS_PT)])
        plsc.subcore_barrier()

        def body(j, carry):
            pltpu.sync_copy(tab.at[idx_s.at[j]], gbuf)
            pltpu.sync_copy(gbuf, acc.at[idx_d.at[j]], add=True)
            return carry

        lax.fori_loop(0, NCHUNK, body, 0)
        plsc.subcore_barrier()
        pltpu.sync_copy(acc.at[pl.ds(s * ROWS_PT, ROWS_PT)],
                        agg_hbm.at[cb].at[pl.ds(s * ROWS_PT, ROWS_PT)])


def _tc_input(x, W, b2):
    def body(x_ref, w_ref, b_ref, o_ref):
        o_ref[...] = jnp.maximum(
            jnp.dot(x_ref[...], w_ref[...],
                    preferred_element_type=jnp.float32) + b_ref[...], 0.0)

    return pl.pallas_call(
        body,
        grid=(NRB,),
        in_specs=[
            pl.BlockSpec((RB, D_IN), lambda r: (r, 0)),
            pl.BlockSpec((D_IN, D_H), lambda r: (0, 0)),
            pl.BlockSpec((1, D_H), lambda r: (0, 0)),
        ],
        out_specs=pl.BlockSpec((RB, D_H), lambda r: (r, 0)),
        out_shape=jax.ShapeDtypeStruct((N, D_H), jnp.float32),
    )(x, W, b2)


def _tc_project(h, W, deg_col):
    def body(h_ref, w_ref, d_ref, o_ref):
        dinv = lax.rsqrt(d_ref[...])
        o_ref[...] = (jnp.dot(h_ref[...], w_ref[...],
                              preferred_element_type=jnp.float32) * dinv)[None]

    return pl.pallas_call(
        body,
        grid=(NRB, CB),
        in_specs=[
            pl.BlockSpec((RB, D_H), lambda r, cb: (r, 0)),
            pl.BlockSpec((D_H, 128), lambda r, cb: (0, cb)),
            pl.BlockSpec((RB, 1), lambda r, cb: (r, 0)),
        ],
        out_specs=pl.BlockSpec((1, RB, 128), lambda r, cb: (cb, r, 0)),
        out_shape=jax.ShapeDtypeStruct((CB, NPAD, 128), jnp.float32),
    )(h, W, deg_col)


def _tc_bn_relu_res(agg, deg_col, b2, g2, be2, hprev):
    def body(agg_ref, d_ref, b_ref, g_ref, be_ref, hp_ref, o_ref,
             mu_ref, sd_ref):
        p = pl.program_id(0)
        cb = pl.program_id(1)
        rb = pl.program_id(2)
        dinv = lax.rsqrt(d_ref[...])
        pre = agg_ref[0] * dinv + b_ref[...]

        @pl.when(p == 0)
        def _():
            @pl.when(rb == 0)
            def _():
                mu_ref[pl.ds(cb, 1)] = jnp.zeros((1, 128), jnp.float32)
                sd_ref[pl.ds(cb, 1)] = jnp.zeros((1, 128), jnp.float32)

            mu_ref[pl.ds(cb, 1)] += jnp.sum(pre, axis=0, keepdims=True)
            sd_ref[pl.ds(cb, 1)] += jnp.sum(pre * pre, axis=0, keepdims=True)

            @pl.when(rb == NRB - 1)
            def _():
                mu = mu_ref[pl.ds(cb, 1)] * (1.0 / N)
                var = sd_ref[pl.ds(cb, 1)] * (1.0 / N) - mu * mu
                mu_ref[pl.ds(cb, 1)] = mu
                sd_ref[pl.ds(cb, 1)] = lax.rsqrt(var + EPS)

        @pl.when(p == 1)
        def _():
            mu = mu_ref[pl.ds(cb, 1)]
            rstd = sd_ref[pl.ds(cb, 1)]
            o_ref[...] = jnp.maximum(
                (pre - mu) * rstd * g_ref[...] + be_ref[...], 0.0) + hp_ref[...]

    return pl.pallas_call(
        body,
        grid=(2, CB, NRB),
        in_specs=[
            pl.BlockSpec((1, RB, 128), lambda p, cb, rb: (cb, rb, 0)),
            pl.BlockSpec((RB, 1), lambda p, cb, rb: (rb, 0)),
            pl.BlockSpec((1, 128), lambda p, cb, rb: (0, cb)),
            pl.BlockSpec((1, 128), lambda p, cb, rb: (0, cb)),
            pl.BlockSpec((1, 128), lambda p, cb, rb: (0, cb)),
            pl.BlockSpec((RB, 128), lambda p, cb, rb: (rb, cb)),
        ],
        out_specs=pl.BlockSpec((RB, 128), lambda p, cb, rb: (rb, cb)),
        out_shape=jax.ShapeDtypeStruct((N, D_H), jnp.float32),
        scratch_shapes=[
            pltpu.VMEM((CB, 128), jnp.float32),
            pltpu.VMEM((CB, 128), jnp.float32),
        ],
    )(agg, deg_col, b2, g2, be2, hprev)


def _tc_output(h, Wp, bp):
    def body(h_ref, w_ref, b_ref, o_ref):
        o_ref[...] = jnp.dot(h_ref[...], w_ref[...],
                             preferred_element_type=jnp.float32) + b_ref[...]

    return pl.pallas_call(
        body,
        grid=(NRB,),
        in_specs=[
            pl.BlockSpec((RB, D_H), lambda r: (r, 0)),
            pl.BlockSpec((D_H, 128), lambda r: (0, 0)),
            pl.BlockSpec((1, 128), lambda r: (0, 0)),
        ],
        out_specs=pl.BlockSpec((RB, 128), lambda r: (r, 0)),
        out_shape=jax.ShapeDtypeStruct((N, 128), jnp.float32),
    )(h, Wp, bp)


def kernel(x, edge_index, W_in, b_in, W1, b1, g1, beta1, W2, b2, g2, beta2,
           W3, b3, g3, beta3, W_out, b_out):
    pad = jnp.full((EPAD - E,), N, dtype=jnp.int32)
    src_r = jnp.concatenate([edge_index[0], pad]).reshape(NS, NCHUNK, CHUNK)
    dst_r = jnp.concatenate([edge_index[1], pad]).reshape(NS, NCHUNK, CHUNK)

    deg_col = _sc_degree(dst_r).reshape(NPAD, 1)
    h = _tc_input(x, W_in, b_in.reshape(1, D_H))
    for (W, b, g, be) in ((W1, b1, g1, beta1), (W2, b2, g2, beta2),
                          (W3, b3, g3, beta3)):
        Ht = _tc_project(h, W, deg_col)
        agg = _sc_aggregate(Ht, src_r, dst_r)
        h = _tc_bn_relu_res(agg, deg_col, b.reshape(1, D_H),
                            g.reshape(1, D_H), be.reshape(1, D_H), h)

    Wp = jnp.pad(W_out, ((0, 0), (0, 128 - D_OUT)))
    bp = jnp.pad(b_out, (0, 128 - D_OUT)).reshape(1, 128)
    return _tc_output(h, Wp, bp)[:, :D_OUT]


# speedup vs baseline: 4.6449x; 1.0886x over previous
import functools

import jax
import jax.numpy as jnp
from jax import lax
from jax.experimental import pallas as pl
from jax.experimental.pallas import tpu as pltpu
from jax.experimental.pallas import tpu_sc as plsc

N = 10000
E = 160000
D_IN = 256
D_H = 512
D_OUT = 8

NC, NS, L = 2, 16, 16
NPAD = 10240
ROWS_PT = NPAD // NS
CHUNK = 128
EPT = 10240
NCHUNK = EPT // CHUNK
EPAD = NS * EPT
CB = D_H // 128
RB = 400
NRB = N // RB
EPS = 1e-5

_mesh = plsc.VectorSubcoreMesh(core_axis_name="c", subcore_axis_name="s")


@functools.partial(
    pl.kernel,
    out_type=jax.ShapeDtypeStruct((NPAD,), jnp.float32),
    mesh=_mesh,
    scratch_types=[
        pltpu.VMEM_SHARED((NPAD,), jnp.float32),
        pltpu.VMEM((ROWS_PT,), jnp.float32),
        pltpu.VMEM((NCHUNK, CHUNK), jnp.int32),
    ],
)
def _sc_degree(dst_hbm, deg_hbm, hist, ones_v, idx_v):
    c = lax.axis_index("c")
    s = lax.axis_index("s")

    @pl.when(c == 0)
    def _():
        def fill(i, carry):
            ones_v[pl.ds(i * L, L)] = jnp.ones((L,), jnp.float32)
            return carry

        lax.fori_loop(0, ROWS_PT // L, fill, 0)
        pltpu.sync_copy(dst_hbm.at[s], idx_v)
        pltpu.sync_copy(ones_v, hist.at[pl.ds(s * ROWS_PT, ROWS_PT)])
        plsc.subcore_barrier()

        def body(j, carry):
            pltpu.sync_copy(ones_v.at[pl.ds(0, CHUNK)],
                            hist.at[idx_v.at[j]], add=True)
            return carry

        lax.fori_loop(0, NCHUNK, body, 0)
        plsc.subcore_barrier()
        pltpu.sync_copy(hist.at[pl.ds(s * ROWS_PT, ROWS_PT)],
                        deg_hbm.at[pl.ds(s * ROWS_PT, ROWS_PT)])


NBUF = 2
HALF = NCHUNK // 2
NT = HALF // NBUF


@functools.partial(
    pl.kernel,
    out_type=jax.ShapeDtypeStruct((CB, NPAD, 128), jnp.float32),
    mesh=_mesh,
    scratch_types=[
        pltpu.VMEM_SHARED((NPAD, 128), jnp.float32),
        pltpu.VMEM((HALF, CHUNK), jnp.int32),
        pltpu.VMEM((HALF, CHUNK), jnp.int32),
        [pltpu.VMEM((CHUNK, 128), jnp.float32)] * NBUF,
        [pltpu.SemaphoreType.DMA] * NBUF,
        [pltpu.SemaphoreType.DMA] * NBUF,
    ],
)
def _sc_aggregate(tab_hbm, src_hbm, dst_hbm, agg_hbm, acc, idx_s, idx_d,
                  gbuf, gsem, ssem):
    c = lax.axis_index("c")
    s = lax.axis_index("s")
    for cbl in range(CB // NC):
        cb = c * (CB // NC) + cbl
        tab = tab_hbm.at[cb]
        pltpu.sync_copy(tab.at[pl.ds(s * ROWS_PT, ROWS_PT)],
                        acc.at[pl.ds(s * ROWS_PT, ROWS_PT)])
        plsc.subcore_barrier()

        for h in range(2):
            pltpu.sync_copy(src_hbm.at[s].at[pl.ds(h * HALF, HALF)], idx_s)
            pltpu.sync_copy(dst_hbm.at[s].at[pl.ds(h * HALF, HALF)], idx_d)
            for b in range(NBUF):
                pltpu.async_copy(tab.at[idx_s.at[b]], gbuf[b], gsem[b])

            @pl.loop(0, NT)
            def _(t):
                scat = []
                for b in range(NBUF):
                    j = t * NBUF + b
                    pltpu.make_async_copy(tab.at[idx_s.at[j]],
                                          gbuf[b], gsem[b]).wait()
                    scat.append(pltpu.async_copy(
                        gbuf[b], acc.at[idx_d.at[j]], ssem[b], add=True))
                for b in range(NBUF):
                    j = t * NBUF + b
                    scat[b].wait()

                    @pl.when(t < NT - 1)
                    def _():
                        pltpu.async_copy(tab.at[idx_s.at[j + NBUF]],
                                         gbuf[b], gsem[b])

        plsc.subcore_barrier()
        pltpu.sync_copy(acc.at[pl.ds(s * ROWS_PT, ROWS_PT)],
                        agg_hbm.at[cb].at[pl.ds(s * ROWS_PT, ROWS_PT)])


def _tc_input(x, W, b2):
    def body(x_ref, w_ref, b_ref, o_ref):
        o_ref[...] = jnp.maximum(
            jnp.dot(x_ref[...], w_ref[...],
                    preferred_element_type=jnp.float32) + b_ref[...], 0.0)

    return pl.pallas_call(
        body,
        grid=(NRB,),
        in_specs=[
            pl.BlockSpec((RB, D_IN), lambda r: (r, 0)),
            pl.BlockSpec((D_IN, D_H), lambda r: (0, 0)),
            pl.BlockSpec((1, D_H), lambda r: (0, 0)),
        ],
        out_specs=pl.BlockSpec((RB, D_H), lambda r: (r, 0)),
        out_shape=jax.ShapeDtypeStruct((N, D_H), jnp.float32),
    )(x, W, b2)


def _tc_project(h, W, deg_col):
    def body(h_ref, w_ref, d_ref, o_ref):
        dinv = lax.rsqrt(d_ref[...])
        o_ref[...] = (jnp.dot(h_ref[...], w_ref[...],
                              preferred_element_type=jnp.float32) * dinv)[None]

    return pl.pallas_call(
        body,
        grid=(NRB, CB),
        in_specs=[
            pl.BlockSpec((RB, D_H), lambda r, cb: (r, 0)),
            pl.BlockSpec((D_H, 128), lambda r, cb: (0, cb)),
            pl.BlockSpec((RB, 1), lambda r, cb: (r, 0)),
        ],
        out_specs=pl.BlockSpec((1, RB, 128), lambda r, cb: (cb, r, 0)),
        out_shape=jax.ShapeDtypeStruct((CB, NPAD, 128), jnp.float32),
    )(h, W, deg_col)


def _tc_bn_relu_res(agg, deg_col, b2, g2, be2, hprev):
    def body(agg_ref, d_ref, b_ref, g_ref, be_ref, hp_ref, o_ref,
             mu_ref, sd_ref):
        p = pl.program_id(0)
        cb = pl.program_id(1)
        rb = pl.program_id(2)
        dinv = lax.rsqrt(d_ref[...])
        pre = agg_ref[0] * dinv + b_ref[...]

        @pl.when(p == 0)
        def _():
            @pl.when(rb == 0)
            def _():
                mu_ref[pl.ds(cb, 1)] = jnp.zeros((1, 128), jnp.float32)
                sd_ref[pl.ds(cb, 1)] = jnp.zeros((1, 128), jnp.float32)

            mu_ref[pl.ds(cb, 1)] += jnp.sum(pre, axis=0, keepdims=True)
            sd_ref[pl.ds(cb, 1)] += jnp.sum(pre * pre, axis=0, keepdims=True)

            @pl.when(rb == NRB - 1)
            def _():
                mu = mu_ref[pl.ds(cb, 1)] * (1.0 / N)
                var = sd_ref[pl.ds(cb, 1)] * (1.0 / N) - mu * mu
                mu_ref[pl.ds(cb, 1)] = mu
                sd_ref[pl.ds(cb, 1)] = lax.rsqrt(var + EPS)

        @pl.when(p == 1)
        def _():
            mu = mu_ref[pl.ds(cb, 1)]
            rstd = sd_ref[pl.ds(cb, 1)]
            o_ref[...] = jnp.maximum(
                (pre - mu) * rstd * g_ref[...] + be_ref[...], 0.0) + hp_ref[...]

    return pl.pallas_call(
        body,
        grid=(2, CB, NRB),
        in_specs=[
            pl.BlockSpec((1, RB, 128), lambda p, cb, rb: (cb, rb, 0)),
            pl.BlockSpec((RB, 1), lambda p, cb, rb: (rb, 0)),
            pl.BlockSpec((1, 128), lambda p, cb, rb: (0, cb)),
            pl.BlockSpec((1, 128), lambda p, cb, rb: (0, cb)),
            pl.BlockSpec((1, 128), lambda p, cb, rb: (0, cb)),
            pl.BlockSpec((RB, 128), lambda p, cb, rb: (rb, cb)),
        ],
        out_specs=pl.BlockSpec((RB, 128), lambda p, cb, rb: (rb, cb)),
        out_shape=jax.ShapeDtypeStruct((N, D_H), jnp.float32),
        scratch_shapes=[
            pltpu.VMEM((CB, 128), jnp.float32),
            pltpu.VMEM((CB, 128), jnp.float32),
        ],
    )(agg, deg_col, b2, g2, be2, hprev)


def _tc_output(h, Wp, bp):
    def body(h_ref, w_ref, b_ref, o_ref):
        o_ref[...] = jnp.dot(h_ref[...], w_ref[...],
                             preferred_element_type=jnp.float32) + b_ref[...]

    return pl.pallas_call(
        body,
        grid=(NRB,),
        in_specs=[
            pl.BlockSpec((RB, D_H), lambda r: (r, 0)),
            pl.BlockSpec((D_H, 128), lambda r: (0, 0)),
            pl.BlockSpec((1, 128), lambda r: (0, 0)),
        ],
        out_specs=pl.BlockSpec((RB, 128), lambda r: (r, 0)),
        out_shape=jax.ShapeDtypeStruct((N, 128), jnp.float32),
    )(h, Wp, bp)


def kernel(x, edge_index, W_in, b_in, W1, b1, g1, beta1, W2, b2, g2, beta2,
           W3, b3, g3, beta3, W_out, b_out):
    pad = jnp.full((EPAD - E,), N, dtype=jnp.int32)
    src_r = jnp.concatenate([edge_index[0], pad]).reshape(NS, NCHUNK, CHUNK)
    dst_r = jnp.concatenate([edge_index[1], pad]).reshape(NS, NCHUNK, CHUNK)

    deg_col = _sc_degree(dst_r).reshape(NPAD, 1)
    h = _tc_input(x, W_in, b_in.reshape(1, D_H))
    for (W, b, g, be) in ((W1, b1, g1, beta1), (W2, b2, g2, beta2),
                          (W3, b3, g3, beta3)):
        Ht = _tc_project(h, W, deg_col)
        agg = _sc_aggregate(Ht, src_r, dst_r)
        h = _tc_bn_relu_res(agg, deg_col, b.reshape(1, D_H),
                            g.reshape(1, D_H), be.reshape(1, D_H), h)

    Wp = jnp.pad(W_out, ((0, 0), (0, 128 - D_OUT)))
    bp = jnp.pad(b_out, (0, 128 - D_OUT)).reshape(1, 128)
    return _tc_output(h, Wp, bp)[:, :D_OUT]


# speedup vs baseline: 5.4184x; 1.1665x over previous
import functools

import jax
import jax.numpy as jnp
from jax import lax
from jax.experimental import pallas as pl
from jax.experimental.pallas import tpu as pltpu
from jax.experimental.pallas import tpu_sc as plsc

N = 10000
E = 160000
D_IN = 256
D_H = 512
D_OUT = 8

NC, NS, L = 2, 16, 16
NPAD = 10240
ROWS_PT = NPAD // NS
CHUNK = 128
EPT = 10240
NCHUNK = EPT // CHUNK
EPAD = NS * EPT
CB = D_H // 128
RB = 400
NRB = N // RB
EPS = 1e-5

_mesh = plsc.VectorSubcoreMesh(core_axis_name="c", subcore_axis_name="s")


@functools.partial(
    pl.kernel,
    out_type=jax.ShapeDtypeStruct((NPAD,), jnp.float32),
    mesh=_mesh,
    scratch_types=[
        pltpu.VMEM_SHARED((NPAD,), jnp.float32),
        pltpu.VMEM((ROWS_PT,), jnp.float32),
        pltpu.VMEM((NCHUNK, CHUNK), jnp.int32),
    ],
)
def _sc_degree(dst_hbm, deg_hbm, hist, ones_v, idx_v):
    c = lax.axis_index("c")
    s = lax.axis_index("s")

    @pl.when(c == 0)
    def _():
        def fill(i, carry):
            ones_v[pl.ds(i * L, L)] = jnp.ones((L,), jnp.float32)
            return carry

        lax.fori_loop(0, ROWS_PT // L, fill, 0)
        pltpu.sync_copy(dst_hbm.at[s], idx_v)
        pltpu.sync_copy(ones_v, hist.at[pl.ds(s * ROWS_PT, ROWS_PT)])
        plsc.subcore_barrier()

        def body(j, carry):
            pltpu.sync_copy(ones_v.at[pl.ds(0, CHUNK)],
                            hist.at[idx_v.at[j]], add=True)
            return carry

        lax.fori_loop(0, NCHUNK, body, 0)
        plsc.subcore_barrier()
        pltpu.sync_copy(hist.at[pl.ds(s * ROWS_PT, ROWS_PT)],
                        deg_hbm.at[pl.ds(s * ROWS_PT, ROWS_PT)])


NBUF = 2
HALF = NCHUNK // 2
NT = HALF // NBUF


@functools.partial(
    pl.kernel,
    out_type=jax.ShapeDtypeStruct((NPAD, D_H), jnp.float32),
    mesh=_mesh,
    scratch_types=[
        pltpu.VMEM_SHARED((NPAD, 128), jnp.float32),
        pltpu.VMEM((HALF, CHUNK), jnp.int32),
        pltpu.VMEM((HALF, CHUNK), jnp.int32),
        [pltpu.VMEM((CHUNK, 128), jnp.float32)] * NBUF,
        [pltpu.SemaphoreType.DMA] * NBUF,
        [pltpu.SemaphoreType.DMA] * NBUF,
    ],
)
def _sc_aggregate(tab_hbm, src_hbm, dst_hbm, agg_hbm, acc, idx_s, idx_d,
                  gbuf, gsem, ssem):
    c = lax.axis_index("c")
    s = lax.axis_index("s")
    for cbl in range(CB // NC):
        cb = c * (CB // NC) + cbl
        tab = tab_hbm.at[:, pl.ds(cb * 128, 128)]
        pltpu.sync_copy(tab.at[pl.ds(s * ROWS_PT, ROWS_PT)],
                        acc.at[pl.ds(s * ROWS_PT, ROWS_PT)])
        plsc.subcore_barrier()

        for h in range(2):
            pltpu.sync_copy(src_hbm.at[s].at[pl.ds(h * HALF, HALF)], idx_s)
            pltpu.sync_copy(dst_hbm.at[s].at[pl.ds(h * HALF, HALF)], idx_d)
            for b in range(NBUF):
                pltpu.async_copy(tab.at[idx_s.at[b]], gbuf[b], gsem[b])

            @pl.loop(0, NT)
            def _(t):
                scat = []
                for b in range(NBUF):
                    j = t * NBUF + b
                    pltpu.make_async_copy(tab.at[idx_s.at[j]],
                                          gbuf[b], gsem[b]).wait()
                    scat.append(pltpu.async_copy(
                        gbuf[b], acc.at[idx_d.at[j]], ssem[b], add=True))
                for b in range(NBUF):
                    j = t * NBUF + b
                    scat[b].wait()

                    @pl.when(t < NT - 1)
                    def _():
                        pltpu.async_copy(tab.at[idx_s.at[j + NBUF]],
                                         gbuf[b], gsem[b])

        plsc.subcore_barrier()
        pltpu.sync_copy(
            acc.at[pl.ds(s * ROWS_PT, ROWS_PT)],
            agg_hbm.at[pl.ds(s * ROWS_PT, ROWS_PT), pl.ds(cb * 128, 128)])


def _tc_input(x, W, b2):
    def body(x_ref, w_ref, b_ref, o_ref):
        o_ref[...] = jnp.maximum(
            jnp.dot(x_ref[...], w_ref[...],
                    preferred_element_type=jnp.float32) + b_ref[...], 0.0)

    return pl.pallas_call(
        body,
        grid=(NRB,),
        in_specs=[
            pl.BlockSpec((RB, D_IN), lambda r: (r, 0)),
            pl.BlockSpec((D_IN, D_H), lambda r: (0, 0)),
            pl.BlockSpec((1, D_H), lambda r: (0, 0)),
        ],
        out_specs=pl.BlockSpec((RB, D_H), lambda r: (r, 0)),
        out_shape=jax.ShapeDtypeStruct((N, D_H), jnp.float32),
    )(x, W, b2)


def _tc_project(h, W, deg_col):
    def body(h_ref, w_ref, d_ref, o_ref):
        dinv = lax.rsqrt(d_ref[...])
        o_ref[...] = jnp.dot(h_ref[...], w_ref[...],
                             preferred_element_type=jnp.float32) * dinv

    return pl.pallas_call(
        body,
        grid=(NRB,),
        in_specs=[
            pl.BlockSpec((RB, D_H), lambda r: (r, 0)),
            pl.BlockSpec((D_H, D_H), lambda r: (0, 0)),
            pl.BlockSpec((RB, 1), lambda r: (r, 0)),
        ],
        out_specs=pl.BlockSpec((RB, D_H), lambda r: (r, 0)),
        out_shape=jax.ShapeDtypeStruct((NPAD, D_H), jnp.float32),
    )(h, W, deg_col)


def _tc_bn_stats(agg, deg_col, b2):
    def body(agg_ref, d_ref, b_ref, o_ref, sum_ref, sq_ref):
        rb = pl.program_id(0)
        dinv = lax.rsqrt(d_ref[...])
        pre = agg_ref[...] * dinv + b_ref[...]

        @pl.when(rb == 0)
        def _():
            sum_ref[...] = jnp.zeros((1, D_H), jnp.float32)
            sq_ref[...] = jnp.zeros((1, D_H), jnp.float32)

        sum_ref[...] += jnp.sum(pre, axis=0, keepdims=True)
        sq_ref[...] += jnp.sum(pre * pre, axis=0, keepdims=True)

        @pl.when(rb == NRB - 1)
        def _():
            mu = sum_ref[...] * (1.0 / N)
            var = sq_ref[...] * (1.0 / N) - mu * mu
            o_ref[...] = jnp.concatenate([mu, lax.rsqrt(var + EPS)], axis=0)

    return pl.pallas_call(
        body,
        grid=(NRB,),
        in_specs=[
            pl.BlockSpec((RB, D_H), lambda r: (r, 0)),
            pl.BlockSpec((RB, 1), lambda r: (r, 0)),
            pl.BlockSpec((1, D_H), lambda r: (0, 0)),
        ],
        out_specs=pl.BlockSpec((2, D_H), lambda r: (0, 0)),
        out_shape=jax.ShapeDtypeStruct((2, D_H), jnp.float32),
        scratch_shapes=[
            pltpu.VMEM((1, D_H), jnp.float32),
            pltpu.VMEM((1, D_H), jnp.float32),
        ],
    )(agg, deg_col, b2)


def _tc_bn_apply(agg, deg_col, b2, g2, be2, stats, hprev):
    def body(agg_ref, d_ref, b_ref, g_ref, be_ref, st_ref, hp_ref, o_ref):
        dinv = lax.rsqrt(d_ref[...])
        pre = agg_ref[...] * dinv + b_ref[...]
        mu = st_ref[pl.ds(0, 1)]
        rstd = st_ref[pl.ds(1, 1)]
        o_ref[...] = jnp.maximum(
            (pre - mu) * rstd * g_ref[...] + be_ref[...], 0.0) + hp_ref[...]

    return pl.pallas_call(
        body,
        grid=(NRB,),
        in_specs=[
            pl.BlockSpec((RB, D_H), lambda r: (r, 0)),
            pl.BlockSpec((RB, 1), lambda r: (r, 0)),
            pl.BlockSpec((1, D_H), lambda r: (0, 0)),
            pl.BlockSpec((1, D_H), lambda r: (0, 0)),
            pl.BlockSpec((1, D_H), lambda r: (0, 0)),
            pl.BlockSpec((2, D_H), lambda r: (0, 0)),
            pl.BlockSpec((RB, D_H), lambda r: (r, 0)),
        ],
        out_specs=pl.BlockSpec((RB, D_H), lambda r: (r, 0)),
        out_shape=jax.ShapeDtypeStruct((N, D_H), jnp.float32),
    )(agg, deg_col, b2, g2, be2, stats, hprev)


def _tc_output(h, Wp, bp):
    def body(h_ref, w_ref, b_ref, o_ref):
        o_ref[...] = jnp.dot(h_ref[...], w_ref[...],
                             preferred_element_type=jnp.float32) + b_ref[...]

    return pl.pallas_call(
        body,
        grid=(NRB,),
        in_specs=[
            pl.BlockSpec((RB, D_H), lambda r: (r, 0)),
            pl.BlockSpec((D_H, 128), lambda r: (0, 0)),
            pl.BlockSpec((1, 128), lambda r: (0, 0)),
        ],
        out_specs=pl.BlockSpec((RB, 128), lambda r: (r, 0)),
        out_shape=jax.ShapeDtypeStruct((N, 128), jnp.float32),
    )(h, Wp, bp)


def kernel(x, edge_index, W_in, b_in, W1, b1, g1, beta1, W2, b2, g2, beta2,
           W3, b3, g3, beta3, W_out, b_out):
    pad = jnp.full((EPAD - E,), N, dtype=jnp.int32)
    src_r = jnp.concatenate([edge_index[0], pad]).reshape(NS, NCHUNK, CHUNK)
    dst_r = jnp.concatenate([edge_index[1], pad]).reshape(NS, NCHUNK, CHUNK)

    deg_col = _sc_degree(dst_r).reshape(NPAD, 1)
    h = _tc_input(x, W_in, b_in.reshape(1, D_H))
    for (W, b, g, be) in ((W1, b1, g1, beta1), (W2, b2, g2, beta2),
                          (W3, b3, g3, beta3)):
        Ht = _tc_project(h, W, deg_col)
        agg = _sc_aggregate(Ht, src_r, dst_r)
        b2 = b.reshape(1, D_H)
        stats = _tc_bn_stats(agg, deg_col, b2)
        h = _tc_bn_apply(agg, deg_col, b2, g.reshape(1, D_H),
                         be.reshape(1, D_H), stats, h)

    Wp = jnp.pad(W_out, ((0, 0), (0, 128 - D_OUT)))
    bp = jnp.pad(b_out, (0, 128 - D_OUT)).reshape(1, 128)
    return _tc_output(h, Wp, bp)[:, :D_OUT]


# speedup vs baseline: 5.5697x; 1.0279x over previous
import functools

import jax
import jax.numpy as jnp
from jax import lax
from jax.experimental import pallas as pl
from jax.experimental.pallas import tpu as pltpu
from jax.experimental.pallas import tpu_sc as plsc

N = 10000
E = 160000
D_IN = 256
D_H = 512
D_OUT = 8

NC, NS, L = 2, 16, 16
NPAD = 10240
ROWS_PT = NPAD // NS
CHUNK = 128
EPT = 10240
NCHUNK = EPT // CHUNK
EPAD = NS * EPT
CB = D_H // 128
RB = 400
NRB = N // RB
EPS = 1e-5

_mesh = plsc.VectorSubcoreMesh(core_axis_name="c", subcore_axis_name="s")


@functools.partial(
    pl.kernel,
    out_type=jax.ShapeDtypeStruct((NPAD,), jnp.float32),
    mesh=_mesh,
    scratch_types=[
        pltpu.VMEM_SHARED((NPAD,), jnp.float32),
        pltpu.VMEM((ROWS_PT,), jnp.float32),
        pltpu.VMEM((NCHUNK, CHUNK), jnp.int32),
    ],
)
def _sc_degree(dst_hbm, deg_hbm, hist, ones_v, idx_v):
    c = lax.axis_index("c")
    s = lax.axis_index("s")

    @pl.when(c == 0)
    def _():
        def fill(i, carry):
            ones_v[pl.ds(i * L, L)] = jnp.ones((L,), jnp.float32)
            return carry

        lax.fori_loop(0, ROWS_PT // L, fill, 0)
        pltpu.sync_copy(dst_hbm.at[s], idx_v)
        pltpu.sync_copy(ones_v, hist.at[pl.ds(s * ROWS_PT, ROWS_PT)])
        plsc.subcore_barrier()

        def body(j, carry):
            pltpu.sync_copy(ones_v.at[pl.ds(0, CHUNK)],
                            hist.at[idx_v.at[j]], add=True)
            return carry

        lax.fori_loop(0, NCHUNK, body, 0)
        plsc.subcore_barrier()
        pltpu.sync_copy(hist.at[pl.ds(s * ROWS_PT, ROWS_PT)],
                        deg_hbm.at[pl.ds(s * ROWS_PT, ROWS_PT)])


NBUF = 2
HALF = NCHUNK // 2
NT = HALF // NBUF
HD = D_H // 2


@functools.partial(
    pl.kernel,
    out_type=jax.ShapeDtypeStruct((NPAD, HD), jnp.float32),
    mesh=_mesh,
    scratch_types=[
        pltpu.VMEM_SHARED((NPAD, 128), jnp.float32),
        pltpu.VMEM((HALF, CHUNK), jnp.int32),
        pltpu.VMEM((HALF, CHUNK), jnp.int32),
        [pltpu.VMEM((CHUNK, 128), jnp.float32)] * NBUF,
        [pltpu.SemaphoreType.DMA] * NBUF,
        [pltpu.SemaphoreType.DMA] * NBUF,
    ],
)
def _sc_aggregate_half(tab_hbm, src_hbm, dst_hbm, agg_hbm, acc, idx_s, idx_d,
                       gbuf, gsem, ssem):
    c = lax.axis_index("c")
    s = lax.axis_index("s")
    tab = tab_hbm.at[:, pl.ds(c * 128, 128)]
    pltpu.sync_copy(tab.at[pl.ds(s * ROWS_PT, ROWS_PT)],
                    acc.at[pl.ds(s * ROWS_PT, ROWS_PT)])
    plsc.subcore_barrier()

    for h in range(2):
        pltpu.sync_copy(src_hbm.at[s].at[pl.ds(h * HALF, HALF)], idx_s)
        pltpu.sync_copy(dst_hbm.at[s].at[pl.ds(h * HALF, HALF)], idx_d)
        for b in range(NBUF):
            pltpu.async_copy(tab.at[idx_s.at[b]], gbuf[b], gsem[b])

        @pl.loop(0, NT)
        def _(t):
            scat = []
            for b in range(NBUF):
                j = t * NBUF + b
                pltpu.make_async_copy(tab.at[idx_s.at[j]],
                                      gbuf[b], gsem[b]).wait()
                scat.append(pltpu.async_copy(
                    gbuf[b], acc.at[idx_d.at[j]], ssem[b], add=True))
            for b in range(NBUF):
                j = t * NBUF + b
                scat[b].wait()

                @pl.when(t < NT - 1)
                def _():
                    pltpu.async_copy(tab.at[idx_s.at[j + NBUF]],
                                     gbuf[b], gsem[b])

    plsc.subcore_barrier()
    pltpu.sync_copy(
        acc.at[pl.ds(s * ROWS_PT, ROWS_PT)],
        agg_hbm.at[pl.ds(s * ROWS_PT, ROWS_PT), pl.ds(c * 128, 128)])


def _tc_input(x, W, b2):
    def body(x_ref, w_ref, b_ref, o_ref):
        o_ref[...] = jnp.maximum(
            jnp.dot(x_ref[...], w_ref[...],
                    preferred_element_type=jnp.float32) + b_ref[...], 0.0)

    return pl.pallas_call(
        body,
        grid=(NRB,),
        in_specs=[
            pl.BlockSpec((RB, D_IN), lambda r: (r, 0)),
            pl.BlockSpec((D_IN, D_H), lambda r: (0, 0)),
            pl.BlockSpec((1, D_H), lambda r: (0, 0)),
        ],
        out_specs=pl.BlockSpec((RB, D_H), lambda r: (r, 0)),
        out_shape=jax.ShapeDtypeStruct((N, D_H), jnp.float32),
    )(x, W, b2)


def _tc_project(h, Wh, deg_col):
    def body(h_ref, w_ref, d_ref, o_ref):
        dinv = lax.rsqrt(d_ref[...])
        o_ref[...] = jnp.dot(h_ref[...], w_ref[...],
                             preferred_element_type=jnp.float32) * dinv

    return pl.pallas_call(
        body,
        grid=(NRB,),
        in_specs=[
            pl.BlockSpec((RB, D_H), lambda r: (r, 0)),
            pl.BlockSpec((D_H, HD), lambda r: (0, 0)),
            pl.BlockSpec((RB, 1), lambda r: (r, 0)),
        ],
        out_specs=pl.BlockSpec((RB, HD), lambda r: (r, 0)),
        out_shape=jax.ShapeDtypeStruct((NPAD, HD), jnp.float32),
    )(h, Wh, deg_col)


def _tc_bn_stats(agg, deg_col, b2):
    def body(agg_ref, d_ref, b_ref, o_ref, sum_ref, sq_ref):
        rb = pl.program_id(0)
        dinv = lax.rsqrt(d_ref[...])
        pre = agg_ref[...] * dinv + b_ref[...]

        @pl.when(rb == 0)
        def _():
            sum_ref[...] = jnp.zeros((1, HD), jnp.float32)
            sq_ref[...] = jnp.zeros((1, HD), jnp.float32)

        sum_ref[...] += jnp.sum(pre, axis=0, keepdims=True)
        sq_ref[...] += jnp.sum(pre * pre, axis=0, keepdims=True)

        @pl.when(rb == NRB - 1)
        def _():
            mu = sum_ref[...] * (1.0 / N)
            var = sq_ref[...] * (1.0 / N) - mu * mu
            o_ref[...] = jnp.concatenate([mu, lax.rsqrt(var + EPS)], axis=0)

    return pl.pallas_call(
        body,
        grid=(NRB,),
        in_specs=[
            pl.BlockSpec((RB, HD), lambda r: (r, 0)),
            pl.BlockSpec((RB, 1), lambda r: (r, 0)),
            pl.BlockSpec((1, HD), lambda r: (0, 0)),
        ],
        out_specs=pl.BlockSpec((2, HD), lambda r: (0, 0)),
        out_shape=jax.ShapeDtypeStruct((2, HD), jnp.float32),
        scratch_shapes=[
            pltpu.VMEM((1, HD), jnp.float32),
            pltpu.VMEM((1, HD), jnp.float32),
        ],
    )(agg, deg_col, b2)


def _tc_bn_apply(a0, a1, deg_col, b2, g2, be2, s0, s1, hprev):
    def body(a0_ref, a1_ref, d_ref, b_ref, g_ref, be_ref, s0_ref, s1_ref,
             hp_ref, o_ref):
        dinv = lax.rsqrt(d_ref[...])
        pre = jnp.concatenate([a0_ref[...], a1_ref[...]], axis=1) * dinv \
            + b_ref[...]
        mu = jnp.concatenate([s0_ref[pl.ds(0, 1)], s1_ref[pl.ds(0, 1)]],
                             axis=1)
        rstd = jnp.concatenate([s0_ref[pl.ds(1, 1)], s1_ref[pl.ds(1, 1)]],
                               axis=1)
        o_ref[...] = jnp.maximum(
            (pre - mu) * rstd * g_ref[...] + be_ref[...], 0.0) + hp_ref[...]

    return pl.pallas_call(
        body,
        grid=(NRB,),
        in_specs=[
            pl.BlockSpec((RB, HD), lambda r: (r, 0)),
            pl.BlockSpec((RB, HD), lambda r: (r, 0)),
            pl.BlockSpec((RB, 1), lambda r: (r, 0)),
            pl.BlockSpec((1, D_H), lambda r: (0, 0)),
            pl.BlockSpec((1, D_H), lambda r: (0, 0)),
            pl.BlockSpec((1, D_H), lambda r: (0, 0)),
            pl.BlockSpec((2, HD), lambda r: (0, 0)),
            pl.BlockSpec((2, HD), lambda r: (0, 0)),
            pl.BlockSpec((RB, D_H), lambda r: (r, 0)),
        ],
        out_specs=pl.BlockSpec((RB, D_H), lambda r: (r, 0)),
        out_shape=jax.ShapeDtypeStruct((N, D_H), jnp.float32),
    )(a0, a1, deg_col, b2, g2, be2, s0, s1, hprev)


def _tc_output(h, Wp, bp):
    def body(h_ref, w_ref, b_ref, o_ref):
        o_ref[...] = jnp.dot(h_ref[...], w_ref[...],
                             preferred_element_type=jnp.float32) + b_ref[...]

    return pl.pallas_call(
        body,
        grid=(NRB,),
        in_specs=[
            pl.BlockSpec((RB, D_H), lambda r: (r, 0)),
            pl.BlockSpec((D_H, 128), lambda r: (0, 0)),
            pl.BlockSpec((1, 128), lambda r: (0, 0)),
        ],
        out_specs=pl.BlockSpec((RB, 128), lambda r: (r, 0)),
        out_shape=jax.ShapeDtypeStruct((N, 128), jnp.float32),
    )(h, Wp, bp)


def kernel(x, edge_index, W_in, b_in, W1, b1, g1, beta1, W2, b2, g2, beta2,
           W3, b3, g3, beta3, W_out, b_out):
    pad = jnp.full((EPAD - E,), N, dtype=jnp.int32)
    src_r = jnp.concatenate([edge_index[0], pad]).reshape(NS, NCHUNK, CHUNK)
    dst_r = jnp.concatenate([edge_index[1], pad]).reshape(NS, NCHUNK, CHUNK)

    deg_col = _sc_degree(dst_r).reshape(NPAD, 1)
    h = _tc_input(x, W_in, b_in.reshape(1, D_H))
    for (W, b, g, be) in ((W1, b1, g1, beta1), (W2, b2, g2, beta2),
                          (W3, b3, g3, beta3)):
        b2 = b.reshape(1, D_H)
        P0 = _tc_project(h, W[:, :HD], deg_col)
        A0 = _sc_aggregate_half(P0, src_r, dst_r)
        P1 = _tc_project(h, W[:, HD:], deg_col)
        A1 = _sc_aggregate_half(P1, src_r, dst_r)
        S0 = _tc_bn_stats(A0, deg_col, b2[:, :HD])
        S1 = _tc_bn_stats(A1, deg_col, b2[:, HD:])
        h = _tc_bn_apply(A0, A1, deg_col, b2, g.reshape(1, D_H),
                         be.reshape(1, D_H), S0, S1, h)

    Wp = jnp.pad(W_out, ((0, 0), (0, 128 - D_OUT)))
    bp = jnp.pad(b_out, (0, 128 - D_OUT)).reshape(1, 128)
    return _tc_output(h, Wp, bp)[:, :D_OUT]
